# baseline (speedup 1.0000x reference)
import functools

import jax
import jax.numpy as jnp
from jax import lax
from jax.experimental import pallas as pl
from jax.experimental.pallas import tpu as pltpu
from jax.experimental.pallas import tpu_sc as plsc

_N = 10000
_E = 320000
_NC = 2
_NS = 16
_NW = _NC * _NS
_EPT = _E // _NW
_CH = 80
_NCHUNK = _EPT // _CH
_RB = 624
_RLAST = _N - (_NS - 1) * _RB

_CHS = 80
_NCHS = 125
_EPTS = _CHS * _NCHS
_NSINK = 64
_NACC = _N + _NSINK


def _sc_mesh():
  return plsc.VectorSubcoreMesh(core_axis_name="c", subcore_axis_name="s")


@functools.cache
def _make_scatter_rows(d):
  @functools.partial(
      pl.kernel,
      out_type=jax.ShapeDtypeStruct((_NC, _N, d), jnp.float32),
      mesh=_sc_mesh(),
      scratch_types=[
          [pltpu.VMEM((_CHS,), jnp.int32) for _ in range(4)],
          [pltpu.VMEM((_CHS,), jnp.int32) for _ in range(4)],
          [pltpu.VMEM((_CHS, d), jnp.float32) for _ in range(4)],
          pltpu.VMEM_SHARED((_NACC, d), jnp.float32),
          [pltpu.SemaphoreType.DMA for _ in range(4)],
          [pltpu.SemaphoreType.DMA for _ in range(4)],
          [pltpu.SemaphoreType.DMA for _ in range(4)],
      ],
  )
  def scatter_rows(src_hbm, dst_hbm, y_hbm, out_hbm, sidxb, didxb, rows,
                   acc, semi, semg, sems):
    c = lax.axis_index("c")
    s = lax.axis_index("s")
    wid = c * _NS + s

    def zrow(i, carry):
      for jj in range(d // 16):
        rows[0][i, pl.ds(jj * 16, 16)] = jnp.zeros((16,), jnp.float32)
      return carry

    lax.fori_loop(0, _CHS, zrow, 0)
    row0 = pl.multiple_of(s * _RB, 8)

    @pl.when(s < _NS - 1)
    def _():
      for k in range(_RB // _CHS):
        pltpu.async_copy(rows[0], acc.at[pl.ds(row0 + k * _CHS, _CHS)],
                         semg[0])
      for k in range(_RB // _CHS):
        pltpu.make_async_copy(rows[0], acc.at[pl.ds(row0, _CHS)],
                              semg[0]).wait()
      if _RB % _CHS:
        pltpu.sync_copy(rows[0].at[pl.ds(0, _RB % _CHS)],
                        acc.at[pl.ds(row0 + (_RB // _CHS) * _CHS,
                                     _RB % _CHS)])

    @pl.when(s == _NS - 1)
    def _():
      for k in range(_RLAST // _CHS):
        pltpu.async_copy(
            rows[0], acc.at[pl.ds((_NS - 1) * _RB + k * _CHS, _CHS)],
            semg[0])
      for k in range(_RLAST // _CHS):
        pltpu.make_async_copy(
            rows[0], acc.at[pl.ds((_NS - 1) * _RB, _CHS)], semg[0]).wait()
      if _RLAST % _CHS:
        pltpu.sync_copy(
            rows[0].at[pl.ds(0, _RLAST % _CHS)],
            acc.at[pl.ds((_NS - 1) * _RB + (_RLAST // _CHS) * _CHS,
                         _RLAST % _CHS)])

    plsc.subcore_barrier()

    ebase = pl.multiple_of(wid * _EPTS, 8)

    def idxload(j, p):
      off = pl.multiple_of(ebase + j * _CHS, 8)
      pltpu.async_copy(src_hbm.at[pl.ds(off, _CHS)], sidxb[p], semi[p])
      pltpu.async_copy(dst_hbm.at[pl.ds(off, _CHS)], didxb[p], semi[p])

    def wait_idx(p):
      pltpu.make_async_copy(src_hbm.at[pl.ds(0, _CHS)], sidxb[p],
                            semi[p]).wait()
      pltpu.make_async_copy(src_hbm.at[pl.ds(0, _CHS)], didxb[p],
                            semi[p]).wait()

    def gather(j, p):
      pltpu.async_copy(y_hbm.at[sidxb[p]], rows[p], semg[p])

    def scatter(j, p):
      pltpu.async_copy(rows[p], acc.at[didxb[p]], sems[p], add=True)

    def wait_rows(sem):
      pltpu.make_async_copy(y_hbm.at[pl.ds(0, _CHS)], rows[0], sem).wait()

    idxload(0, 0)
    idxload(1, 1)
    idxload(2, 2)
    idxload(3, 3)
    wait_idx(0)
    gather(0, 0)

    def gen(j, p, do_load):
      wait_rows(semg[p])
      scatter(j, p)
      if do_load:
        wait_rows(sems[(p + 2) % 4])
        idxload(j + 2, (p + 2) % 4)
      wait_idx((p + 1) % 4)
      gather(j + 1, (p + 1) % 4)

    wait_rows(semg[0])
    scatter(0, 0)
    wait_idx(1)
    gather(1, 1)
    wait_rows(semg[1])
    scatter(1, 1)
    wait_idx(2)
    gather(2, 2)

    def step(i, carry):
      j = 2 + 4 * i
      gen(j, 2, True)
      gen(j + 1, 3, True)
      gen(j + 2, 0, True)
      gen(j + 3, 1, True)
      return carry

    lax.fori_loop(0, 30, step, 0)
    gen(122, 2, True)
    wait_rows(semg[3])
    scatter(123, 3)
    wait_rows(sems[1])
    wait_idx(0)
    gather(124, 0)
    wait_rows(semg[0])
    scatter(124, 0)
    wait_rows(sems[2])
    wait_rows(sems[3])
    wait_rows(sems[0])
    plsc.subcore_barrier()

    @pl.when(s < _NS - 1)
    def _():
      pltpu.sync_copy(acc.at[pl.ds(row0, _RB)], out_hbm.at[c, pl.ds(row0, _RB)])

    @pl.when(s == _NS - 1)
    def _():
      pltpu.sync_copy(acc.at[pl.ds((_NS - 1) * _RB, _RLAST)],
                      out_hbm.at[c, pl.ds((_NS - 1) * _RB, _RLAST)])

  return scatter_rows


@functools.cache
def _make_degree():
  @functools.partial(
      pl.kernel,
      out_type=jax.ShapeDtypeStruct((_NC, _N), jnp.float32),
      mesh=_sc_mesh(),
      scratch_types=[
          pltpu.VMEM((_CH,), jnp.int32),
          pltpu.VMEM((_CH,), jnp.int32),
          pltpu.VMEM((_CH,), jnp.float32),
          pltpu.VMEM((_N,), jnp.float32),
          pltpu.VMEM_SHARED((_N,), jnp.float32),
          pltpu.SemaphoreType.DMA,
          pltpu.SemaphoreType.DMA,
          pltpu.SemaphoreType.DMA,
          pltpu.SemaphoreType.DMA,
      ],
  )
  def degree(dst_hbm, out_hbm, didx0, didx1, ones, zbuf, acc,
             semi0, semi1, sems0, sems1):
    c = lax.axis_index("c")
    s = lax.axis_index("s")
    wid = c * _NS + s

    for jj in range(_CH // 16):
      ones[pl.ds(jj * 16, 16)] = jnp.ones((16,), jnp.float32)

    @pl.when(s == 0)
    def _():
      def zrow(i, carry):
        zbuf[pl.ds(i * 16, 16)] = jnp.zeros((16,), jnp.float32)
        return carry

      lax.fori_loop(0, _N // 16, zrow, 0)
      pltpu.sync_copy(zbuf, acc)

    plsc.subcore_barrier()

    ebase = wid * _EPT

    def load(j, db, si):
      off = pl.multiple_of(ebase + j * _CH, 8)
      pltpu.async_copy(dst_hbm.at[pl.ds(off, _CH)], db, si)

    def scat(db, ss):
      pltpu.async_copy(ones, acc.at[db], ss, add=True)

    def wait_idx(db, sem):
      pltpu.make_async_copy(dst_hbm.at[pl.ds(0, _CH)], db, sem).wait()

    def wait_scat(sem):
      pltpu.make_async_copy(dst_hbm.at[pl.ds(0, _CH)], ones, sem).wait()

    load(0, didx0, semi0)
    wait_idx(didx0, semi0)
    scat(didx0, sems0)
    load(1, didx1, semi1)
    wait_idx(didx1, semi1)
    scat(didx1, sems1)
    wait_scat(sems0)
    load(2, didx0, semi0)

    def body(i, carry):
      a = 2 * i
      wait_idx(didx0, semi0)
      scat(didx0, sems0)
      wait_scat(sems1)
      load(a + 1, didx1, semi1)
      wait_idx(didx1, semi1)
      scat(didx1, sems1)
      wait_scat(sems0)
      load(a + 2, didx0, semi0)
      return carry

    lax.fori_loop(1, (_NCHUNK - 1) // 2, body, 0)
    wait_idx(didx0, semi0)
    scat(didx0, sems0)
    wait_scat(sems1)
    wait_scat(sems0)
    plsc.subcore_barrier()

    @pl.when(s == 0)
    def _():
      pltpu.sync_copy(acc, out_hbm.at[c])

  return degree


_B = 1000


def _tc_mm(x, W):
  din, dout = W.shape

  def body(x_ref, w_ref, u_ref):
    u_ref[...] = jnp.dot(x_ref[...], w_ref[...],
                         preferred_element_type=jnp.float32)

  return pl.pallas_call(
      body,
      grid=(_N // _B,),
      in_specs=[
          pl.BlockSpec((_B, din), lambda i: (i, 0)),
          pl.BlockSpec((din, dout), lambda i: (0, 0)),
      ],
      out_specs=pl.BlockSpec((_B, dout), lambda i: (i, 0)),
      out_shape=jax.ShapeDtypeStruct((_N, dout), jnp.float32),
  )(x, W)


def _tc_scale(u, deg0, deg1):
  dout = u.shape[1]

  def body(u_ref, d0_ref, d1_ref, y_ref, dis_ref):
    dis = lax.rsqrt(d0_ref[...] + d1_ref[...] + 1.0)
    y_ref[...] = dis * u_ref[...]
    dis_ref[...] = dis

  return pl.pallas_call(
      body,
      grid=(_N // _B,),
      in_specs=[
          pl.BlockSpec((_B, dout), lambda i: (i, 0)),
          pl.BlockSpec((_B, 1), lambda i: (i, 0)),
          pl.BlockSpec((_B, 1), lambda i: (i, 0)),
      ],
      out_specs=[
          pl.BlockSpec((_B, dout), lambda i: (i, 0)),
          pl.BlockSpec((_B, 1), lambda i: (i, 0)),
      ],
      out_shape=[
          jax.ShapeDtypeStruct((_N, dout), jnp.float32),
          jax.ShapeDtypeStruct((_N, 1), jnp.float32),
      ],
  )(u, deg0, deg1)


def _tc_mid(s0, s1, y, dis, b, W, act_tanh, cond=None, W2=None, pad_to=None):
  din, dout = W.shape
  arr_w = y.shape[1]
  has_cond = cond is not None
  out_w = pad_to if pad_to is not None else dout

  def body(*refs):
    if has_cond:
      s0_ref, s1_ref, y_ref, dis_ref, b_ref, w_ref, c_ref, w2_ref, o_ref = refs
    else:
      s0_ref, s1_ref, y_ref, dis_ref, b_ref, w_ref, o_ref = refs
    dis = dis_ref[...]
    t = (s0_ref[...] + s1_ref[...] + y_ref[...])[:, :din]
    x = dis * t + b_ref[...]
    if act_tanh:
      x = jnp.tanh(x)
    u = jnp.dot(x, w_ref[...], preferred_element_type=jnp.float32)
    if has_cond:
      u = u + jnp.dot(c_ref[...], w2_ref[...],
                      preferred_element_type=jnp.float32)
    u = dis * u
    if pad_to is not None:
      u = jnp.concatenate(
          [u, jnp.zeros((u.shape[0], out_w - dout), jnp.float32)], axis=1)
    o_ref[...] = u

  in_specs = [
      pl.BlockSpec((_B, arr_w), lambda i: (i, 0)),
      pl.BlockSpec((_B, arr_w), lambda i: (i, 0)),
      pl.BlockSpec((_B, arr_w), lambda i: (i, 0)),
      pl.BlockSpec((_B, 1), lambda i: (i, 0)),
      pl.BlockSpec((1, din), lambda i: (0, 0)),
      pl.BlockSpec((din, dout), lambda i: (0, 0)),
  ]
  args = [s0, s1, y, dis, b, W]
  if has_cond:
    in_specs += [
        pl.BlockSpec((_B, cond.shape[1]), lambda i: (i, 0)),
        pl.BlockSpec((cond.shape[1], dout), lambda i: (0, 0)),
    ]
    args += [cond, W2]

  return pl.pallas_call(
      body,
      grid=(_N // _B,),
      in_specs=in_specs,
      out_specs=pl.BlockSpec((_B, out_w), lambda i: (i, 0)),
      out_shape=jax.ShapeDtypeStruct((_N, out_w), jnp.float32),
  )(*args)


def _tc_final(s0, s1, y, dis, b):
  din = y.shape[1]

  def body(s0_ref, s1_ref, y_ref, dis_ref, b_ref, o_ref):
    o_ref[...] = dis_ref[...] * (s0_ref[...] + s1_ref[...] + y_ref[...]) \
        + b_ref[...]

  return pl.pallas_call(
      body,
      grid=(_N // _B,),
      in_specs=[
          pl.BlockSpec((_B, din), lambda i: (i, 0)),
          pl.BlockSpec((_B, din), lambda i: (i, 0)),
          pl.BlockSpec((_B, din), lambda i: (i, 0)),
          pl.BlockSpec((_B, 1), lambda i: (i, 0)),
          pl.BlockSpec((1, din), lambda i: (0, 0)),
      ],
      out_specs=pl.BlockSpec((_B, din), lambda i: (i, 0)),
      out_shape=jax.ShapeDtypeStruct((_N, din), jnp.float32),
  )(s0, s1, y, dis, b)


def kernel(feature, condition, edge_index, W_e1, b_e1, W_e2, b_e2, W_e3, b_e3,
           W_d1, b_d1, W_d2, b_d2, W_d3, b_d3):
  src = edge_index[0]
  dst = edge_index[1]

  deg = _make_degree()(dst)
  deg0 = deg[0].reshape(_N, 1)
  deg1 = deg[1].reshape(_N, 1)

  scat128 = _make_scatter_rows(128)

  x1 = jnp.concatenate([feature, condition], axis=1)
  u1 = _tc_mm(x1, W_e1)
  y1, dis = _tc_scale(u1, deg0, deg1)

  s = scat128(src, dst, y1)
  y2 = _tc_mid(s[0], s[1], y1, dis, b_e1.reshape(1, -1), W_e2, True)
  s = scat128(src, dst, y2)
  y3 = _tc_mid(s[0], s[1], y2, dis, b_e2.reshape(1, -1), W_e3, True,
               pad_to=128)
  s = scat128(src, dst, y3)
  y4 = _tc_mid(s[0], s[1], y3, dis, b_e3.reshape(1, -1), W_d1[:64], False,
               cond=condition, W2=W_d1[64:])
  s = scat128(src, dst, y4)
  y5 = _tc_mid(s[0], s[1], y4, dis, b_d1.reshape(1, -1), W_d2, True)
  s = scat128(src, dst, y5)
  y6 = _tc_mid(s[0], s[1], y5, dis, b_d2.reshape(1, -1), W_d3, True)
  s = scat128(src, dst, y6)
  return _tc_final(s[0], s[1], y6, dis, b_d3.reshape(1, -1))


# speedup vs baseline: 1.0090x; 1.0090x over previous
import functools

import jax
import jax.numpy as jnp
from jax import lax
from jax.experimental import pallas as pl
from jax.experimental.pallas import tpu as pltpu
from jax.experimental.pallas import tpu_sc as plsc

_N = 10000
_E = 320000
_NC = 2
_NS = 16
_NW = _NC * _NS
_EPT = _E // _NW
_CH = 80
_NCHUNK = _EPT // _CH
_RB = 624
_RLAST = _N - (_NS - 1) * _RB

_CHS = 80
_NCHS = 125
_EPTS = _CHS * _NCHS
_NSINK = 64
_NACC = _N + _NSINK


def _sc_mesh():
  return plsc.VectorSubcoreMesh(core_axis_name="c", subcore_axis_name="s")


@functools.cache
def _make_scatter_rows(d):
  @functools.partial(
      pl.kernel,
      out_type=jax.ShapeDtypeStruct((_NC, _N, d), jnp.float32),
      mesh=_sc_mesh(),
      scratch_types=[
          pltpu.VMEM((_EPTS,), jnp.int32),
          pltpu.VMEM((_NCHS, _CHS), jnp.int32),
          pltpu.VMEM((_CHS, d), jnp.float32),
          pltpu.VMEM((_CHS, d), jnp.float32),
          pltpu.VMEM_SHARED((_NACC, d), jnp.float32),
          pltpu.SemaphoreType.DMA,
          pltpu.SemaphoreType.DMA,
          pltpu.SemaphoreType.DMA,
          pltpu.SemaphoreType.DMA,
      ],
  )
  def scatter_rows(src_hbm, dst3_hbm, y_hbm, out_hbm, sidx, didx, rows0,
                   rows1, acc, semg0, semg1, sems0, sems1):
    c = lax.axis_index("c")
    s = lax.axis_index("s")
    wid = c * _NS + s

    def zrow(i, carry):
      for jj in range(d // 16):
        rows0[i, pl.ds(jj * 16, 16)] = jnp.zeros((16,), jnp.float32)
      return carry

    lax.fori_loop(0, _CHS, zrow, 0)
    row0 = pl.multiple_of(s * _RB, 8)

    ebase = pl.multiple_of(wid * _EPTS, 8)
    pltpu.async_copy(src_hbm.at[pl.ds(ebase, _EPTS)], sidx, semg1)
    pltpu.async_copy(dst3_hbm.at[wid], didx, sems1)

    @pl.when(s < _NS - 1)
    def _():
      for k in range(_RB // _CHS):
        pltpu.async_copy(rows0, acc.at[pl.ds(row0 + k * _CHS, _CHS)], semg0)
      for k in range(_RB // _CHS):
        pltpu.make_async_copy(rows0, acc.at[pl.ds(row0, _CHS)], semg0).wait()
      if _RB % _CHS:
        pltpu.sync_copy(rows0.at[pl.ds(0, _RB % _CHS)],
                        acc.at[pl.ds(row0 + (_RB // _CHS) * _CHS,
                                     _RB % _CHS)])

    @pl.when(s == _NS - 1)
    def _():
      for k in range(_RLAST // _CHS):
        pltpu.async_copy(
            rows0, acc.at[pl.ds((_NS - 1) * _RB + k * _CHS, _CHS)], semg0)
      for k in range(_RLAST // _CHS):
        pltpu.make_async_copy(
            rows0, acc.at[pl.ds((_NS - 1) * _RB, _CHS)], semg0).wait()
      if _RLAST % _CHS:
        pltpu.sync_copy(
            rows0.at[pl.ds(0, _RLAST % _CHS)],
            acc.at[pl.ds((_NS - 1) * _RB + (_RLAST // _CHS) * _CHS,
                         _RLAST % _CHS)])

    pltpu.make_async_copy(src_hbm.at[pl.ds(0, _EPTS)], sidx, semg1).wait()
    pltpu.make_async_copy(dst3_hbm.at[0], didx, sems1).wait()

    plsc.subcore_barrier()

    def gather(j, rb, sg):
      pltpu.async_copy(y_hbm.at[sidx.at[pl.ds(j * _CHS, _CHS)]], rb, sg)

    def scatter(j, rb, ss):
      pltpu.async_copy(rb, acc.at[didx.at[j]], ss, add=True)

    def wait_dma(dst, sem):
      pltpu.make_async_copy(y_hbm.at[pl.ds(0, _CHS)], dst, sem).wait()

    gather(0, rows0, semg0)
    wait_dma(rows0, semg0)
    scatter(0, rows0, sems0)
    gather(1, rows1, semg1)
    wait_dma(rows1, semg1)
    scatter(1, rows1, sems1)
    wait_dma(rows0, sems0)
    gather(2, rows0, semg0)

    def step(i, carry):
      a = 2 * i
      wait_dma(rows0, semg0)
      scatter(a, rows0, sems0)
      wait_dma(rows1, sems1)
      gather(a + 1, rows1, semg1)
      wait_dma(rows1, semg1)
      scatter(a + 1, rows1, sems1)
      wait_dma(rows0, sems0)
      gather(a + 2, rows0, semg0)
      return carry

    lax.fori_loop(1, (_NCHS - 1) // 2, step, 0)
    wait_dma(rows0, semg0)
    scatter(_NCHS - 1, rows0, sems0)
    wait_dma(rows1, sems1)
    wait_dma(rows0, sems0)
    plsc.subcore_barrier()

    @pl.when(s < _NS - 1)
    def _():
      pltpu.sync_copy(acc.at[pl.ds(row0, _RB)], out_hbm.at[c, pl.ds(row0, _RB)])

    @pl.when(s == _NS - 1)
    def _():
      pltpu.sync_copy(acc.at[pl.ds((_NS - 1) * _RB, _RLAST)],
                      out_hbm.at[c, pl.ds((_NS - 1) * _RB, _RLAST)])

  return scatter_rows


@functools.cache
def _make_degree():
  @functools.partial(
      pl.kernel,
      out_type=jax.ShapeDtypeStruct((_NC, _N), jnp.float32),
      mesh=_sc_mesh(),
      scratch_types=[
          pltpu.VMEM((_CH,), jnp.int32),
          pltpu.VMEM((_CH,), jnp.int32),
          pltpu.VMEM((_CH,), jnp.float32),
          pltpu.VMEM((_N,), jnp.float32),
          pltpu.VMEM_SHARED((_N,), jnp.float32),
          pltpu.SemaphoreType.DMA,
          pltpu.SemaphoreType.DMA,
          pltpu.SemaphoreType.DMA,
          pltpu.SemaphoreType.DMA,
      ],
  )
  def degree(dst_hbm, out_hbm, didx0, didx1, ones, zbuf, acc,
             semi0, semi1, sems0, sems1):
    c = lax.axis_index("c")
    s = lax.axis_index("s")
    wid = c * _NS + s

    for jj in range(_CH // 16):
      ones[pl.ds(jj * 16, 16)] = jnp.ones((16,), jnp.float32)

    @pl.when(s == 0)
    def _():
      def zrow(i, carry):
        zbuf[pl.ds(i * 16, 16)] = jnp.zeros((16,), jnp.float32)
        return carry

      lax.fori_loop(0, _N // 16, zrow, 0)
      pltpu.sync_copy(zbuf, acc)

    plsc.subcore_barrier()

    ebase = wid * _EPT

    def load(j, db, si):
      off = pl.multiple_of(ebase + j * _CH, 8)
      pltpu.async_copy(dst_hbm.at[pl.ds(off, _CH)], db, si)

    def scat(db, ss):
      pltpu.async_copy(ones, acc.at[db], ss, add=True)

    def wait_idx(db, sem):
      pltpu.make_async_copy(dst_hbm.at[pl.ds(0, _CH)], db, sem).wait()

    def wait_scat(sem):
      pltpu.make_async_copy(dst_hbm.at[pl.ds(0, _CH)], ones, sem).wait()

    load(0, didx0, semi0)
    wait_idx(didx0, semi0)
    scat(didx0, sems0)
    load(1, didx1, semi1)
    wait_idx(didx1, semi1)
    scat(didx1, sems1)
    wait_scat(sems0)
    load(2, didx0, semi0)

    def body(i, carry):
      a = 2 * i
      wait_idx(didx0, semi0)
      scat(didx0, sems0)
      wait_scat(sems1)
      load(a + 1, didx1, semi1)
      wait_idx(didx1, semi1)
      scat(didx1, sems1)
      wait_scat(sems0)
      load(a + 2, didx0, semi0)
      return carry

    lax.fori_loop(1, (_NCHUNK - 1) // 2, body, 0)
    wait_idx(didx0, semi0)
    scat(didx0, sems0)
    wait_scat(sems1)
    wait_scat(sems0)
    plsc.subcore_barrier()

    @pl.when(s == 0)
    def _():
      pltpu.sync_copy(acc, out_hbm.at[c])

  return degree


_B = 1000


def _tc_mm(x, W):
  din, dout = W.shape

  def body(x_ref, w_ref, u_ref):
    u_ref[...] = jnp.dot(x_ref[...], w_ref[...],
                         preferred_element_type=jnp.float32)

  return pl.pallas_call(
      body,
      grid=(_N // _B,),
      in_specs=[
          pl.BlockSpec((_B, din), lambda i: (i, 0)),
          pl.BlockSpec((din, dout), lambda i: (0, 0)),
      ],
      out_specs=pl.BlockSpec((_B, dout), lambda i: (i, 0)),
      out_shape=jax.ShapeDtypeStruct((_N, dout), jnp.float32),
  )(x, W)


def _tc_scale(u, deg0, deg1):
  dout = u.shape[1]

  def body(u_ref, d0_ref, d1_ref, y_ref, dis_ref):
    dis = lax.rsqrt(d0_ref[...] + d1_ref[...] + 1.0)
    y_ref[...] = dis * u_ref[...]
    dis_ref[...] = dis

  return pl.pallas_call(
      body,
      grid=(_N // _B,),
      in_specs=[
          pl.BlockSpec((_B, dout), lambda i: (i, 0)),
          pl.BlockSpec((_B, 1), lambda i: (i, 0)),
          pl.BlockSpec((_B, 1), lambda i: (i, 0)),
      ],
      out_specs=[
          pl.BlockSpec((_B, dout), lambda i: (i, 0)),
          pl.BlockSpec((_B, 1), lambda i: (i, 0)),
      ],
      out_shape=[
          jax.ShapeDtypeStruct((_N, dout), jnp.float32),
          jax.ShapeDtypeStruct((_N, 1), jnp.float32),
      ],
  )(u, deg0, deg1)


def _tc_mid(s0, s1, y, dis, b, W, act_tanh, cond=None, W2=None, pad_to=None):
  din, dout = W.shape
  arr_w = y.shape[1]
  has_cond = cond is not None
  out_w = pad_to if pad_to is not None else dout

  def body(*refs):
    if has_cond:
      s0_ref, s1_ref, y_ref, dis_ref, b_ref, w_ref, c_ref, w2_ref, o_ref = refs
    else:
      s0_ref, s1_ref, y_ref, dis_ref, b_ref, w_ref, o_ref = refs
    dis = dis_ref[...]
    t = (s0_ref[...] + s1_ref[...] + y_ref[...])[:, :din]
    x = dis * t + b_ref[...]
    if act_tanh:
      x = jnp.tanh(x)
    u = jnp.dot(x, w_ref[...], preferred_element_type=jnp.float32)
    if has_cond:
      u = u + jnp.dot(c_ref[...], w2_ref[...],
                      preferred_element_type=jnp.float32)
    u = dis * u
    if pad_to is not None:
      u = jnp.concatenate(
          [u, jnp.zeros((u.shape[0], out_w - dout), jnp.float32)], axis=1)
    o_ref[...] = u

  in_specs = [
      pl.BlockSpec((_B, arr_w), lambda i: (i, 0)),
      pl.BlockSpec((_B, arr_w), lambda i: (i, 0)),
      pl.BlockSpec((_B, arr_w), lambda i: (i, 0)),
      pl.BlockSpec((_B, 1), lambda i: (i, 0)),
      pl.BlockSpec((1, din), lambda i: (0, 0)),
      pl.BlockSpec((din, dout), lambda i: (0, 0)),
  ]
  args = [s0, s1, y, dis, b, W]
  if has_cond:
    in_specs += [
        pl.BlockSpec((_B, cond.shape[1]), lambda i: (i, 0)),
        pl.BlockSpec((cond.shape[1], dout), lambda i: (0, 0)),
    ]
    args += [cond, W2]

  return pl.pallas_call(
      body,
      grid=(_N // _B,),
      in_specs=in_specs,
      out_specs=pl.BlockSpec((_B, out_w), lambda i: (i, 0)),
      out_shape=jax.ShapeDtypeStruct((_N, out_w), jnp.float32),
  )(*args)


def _tc_final(s0, s1, y, dis, b):
  din = y.shape[1]

  def body(s0_ref, s1_ref, y_ref, dis_ref, b_ref, o_ref):
    o_ref[...] = dis_ref[...] * (s0_ref[...] + s1_ref[...] + y_ref[...]) \
        + b_ref[...]

  return pl.pallas_call(
      body,
      grid=(_N // _B,),
      in_specs=[
          pl.BlockSpec((_B, din), lambda i: (i, 0)),
          pl.BlockSpec((_B, din), lambda i: (i, 0)),
          pl.BlockSpec((_B, din), lambda i: (i, 0)),
          pl.BlockSpec((_B, 1), lambda i: (i, 0)),
          pl.BlockSpec((1, din), lambda i: (0, 0)),
      ],
      out_specs=pl.BlockSpec((_B, din), lambda i: (i, 0)),
      out_shape=jax.ShapeDtypeStruct((_N, din), jnp.float32),
  )(s0, s1, y, dis, b)


def kernel(feature, condition, edge_index, W_e1, b_e1, W_e2, b_e2, W_e3, b_e3,
           W_d1, b_d1, W_d2, b_d2, W_d3, b_d3):
  src = edge_index[0]
  dst = edge_index[1]
  pad = _EPTS - _EPT
  srcp = jnp.concatenate(
      [src.reshape(_NW, _EPT),
       jnp.zeros((_NW, pad), jnp.int32)], axis=1).reshape(-1)
  sink = jnp.broadcast_to(
      _N + (jnp.arange(pad, dtype=jnp.int32) % _NSINK), (_NW, pad))
  dst3 = jnp.concatenate(
      [dst.reshape(_NW, _EPT), sink], axis=1).reshape(_NW, _NCHS, _CHS)

  deg = _make_degree()(dst)
  deg0 = deg[0].reshape(_N, 1)
  deg1 = deg[1].reshape(_N, 1)

  scat128 = _make_scatter_rows(128)

  x1 = jnp.concatenate([feature, condition], axis=1)
  u1 = _tc_mm(x1, W_e1)
  y1, dis = _tc_scale(u1, deg0, deg1)

  s = scat128(srcp, dst3, y1)
  y2 = _tc_mid(s[0], s[1], y1, dis, b_e1.reshape(1, -1), W_e2, True)
  s = scat128(srcp, dst3, y2)
  y3 = _tc_mid(s[0], s[1], y2, dis, b_e2.reshape(1, -1), W_e3, True,
               pad_to=128)
  s = scat128(srcp, dst3, y3)
  y4 = _tc_mid(s[0], s[1], y3, dis, b_e3.reshape(1, -1), W_d1[:64], False,
               cond=condition, W2=W_d1[64:])
  s = scat128(srcp, dst3, y4)
  y5 = _tc_mid(s[0], s[1], y4, dis, b_d1.reshape(1, -1), W_d2, True)
  s = scat128(srcp, dst3, y5)
  y6 = _tc_mid(s[0], s[1], y5, dis, b_d2.reshape(1, -1), W_d3, True)
  s = scat128(srcp, dst3, y6)
  return _tc_final(s[0], s[1], y6, dis, b_d3.reshape(1, -1))
